# Initial kernel scaffold; baseline (speedup 1.0000x reference)
import jax
import jax.numpy as jnp
from jax.experimental import pallas as pl


def kernel(output_stage_one, output_stage_two, mask, ind, target):
    raise NotImplementedError("write your pallas kernel here")


# speedup vs baseline: 1.7634x; 1.7634x over previous
import functools

import jax
import jax.numpy as jnp
from jax import lax
from jax.experimental import pallas as pl
from jax.experimental.pallas import tpu as pltpu
from jax.experimental.pallas import tpu_sc as plsc

B, C, H, W, K = 32, 4, 256, 256, 500
HW = H * W
KP = 512
NJ = KP * C
NCHUNK = NJ // 16

_NC = 2
_NS = 16
NW = _NC * _NS


def _sc_partials(f1, f2, ind, mask_f, tgt):
    mesh = plsc.VectorSubcoreMesh(core_axis_name="c", subcore_axis_name="s")

    @functools.partial(
        pl.kernel,
        mesh=mesh,
        out_type=[
            jax.ShapeDtypeStruct((NW, 16), jnp.float32),
            jax.ShapeDtypeStruct((NW, 16), jnp.float32),
        ],
        scratch_types=[
            pltpu.VMEM((KP,), jnp.int32),
            pltpu.VMEM((KP,), jnp.float32),
            pltpu.VMEM((NJ,), jnp.float32),
            pltpu.VMEM((NJ,), jnp.int32),
            pltpu.VMEM((NJ,), jnp.float32),
            pltpu.VMEM((NJ,), jnp.float32),
            pltpu.VMEM((16,), jnp.float32),
            pltpu.VMEM((16,), jnp.float32),
            pltpu.SemaphoreType.DMA,
            pltpu.SemaphoreType.DMA,
        ],
    )
    def k(f1_hbm, f2_hbm, ind_hbm, mask_hbm, tgt_hbm, acc_out, ms_out,
          ind_v, mask_v, tgt_v, idx_v, p1_v, p2_v, accv, msv, sem1, sem2):
        wid = lax.axis_index("s") * _NC + lax.axis_index("c")
        b = wid

        pltpu.sync_copy(ind_hbm.at[b], ind_v)
        pltpu.sync_copy(mask_hbm.at[b], mask_v)
        pltpu.sync_copy(tgt_hbm.at[b], tgt_v)

        base = b * (C * HW)

        def build(g, _):
            indv = ind_v[pl.ds(g * 16, 16)] + base
            for c in range(C):
                idx_v[pl.ds(c * KP + g * 16, 16)] = indv + c * HW
            return 0

        lax.fori_loop(0, KP // 16, build, 0)

        cp1 = pltpu.async_copy(f1_hbm.at[idx_v], p1_v, sem1)
        cp2 = pltpu.async_copy(f2_hbm.at[idx_v], p2_v, sem2)
        cp1.wait()
        cp2.wait()

        def comp(t, carry):
            acc, ms = carry
            m = mask_v[pl.ds(lax.rem(t, KP // 16) * 16, 16)]
            sl = pl.ds(t * 16, 16)
            e = p1_v[sl] + p2_v[sl] - tgt_v[sl]
            return acc + (m * e) * e, ms + m

        zero = jnp.zeros((16,), jnp.float32)
        acc, ms = lax.fori_loop(0, NCHUNK, comp, (zero, zero))
        accv[:] = acc
        msv[:] = ms
        pltpu.sync_copy(accv, acc_out.at[b])
        pltpu.sync_copy(msv, ms_out.at[b])

    return k(f1, f2, ind, mask_f, tgt)


def _tc_reduce(acc, ms):
    def k(acc_ref, ms_ref, out_ref):
        s1 = jnp.sum(acc_ref[...])
        s2 = jnp.sum(ms_ref[...])
        out_ref[0] = s1 / (s2 + 0.0001)

    return pl.pallas_call(
        k,
        out_shape=jax.ShapeDtypeStruct((1,), jnp.float32),
        out_specs=pl.BlockSpec(memory_space=pltpu.SMEM),
    )(acc, ms)


def kernel(output_stage_one, output_stage_two, mask, ind, target):
    f1 = output_stage_one.reshape(-1)
    f2 = output_stage_two.reshape(-1)
    ind_p = jnp.zeros((B, KP), jnp.int32).at[:, :K].set(ind.astype(jnp.int32))
    mask_f = jnp.zeros((B, KP), jnp.float32).at[:, :K].set(
        mask.astype(jnp.float32))
    tgt_p = jnp.zeros((B, C, KP), jnp.float32).at[:, :, :K].set(
        jnp.transpose(target, (0, 2, 1)))
    tgt_flat = tgt_p.reshape(B, NJ)

    acc, ms = _sc_partials(f1, f2, ind_p, mask_f, tgt_flat)
    return _tc_reduce(acc, ms)[0]


# speedup vs baseline: 1.8095x; 1.0262x over previous
import functools

import jax
import jax.numpy as jnp
from jax import lax
from jax.experimental import pallas as pl
from jax.experimental.pallas import tpu as pltpu
from jax.experimental.pallas import tpu_sc as plsc

B, C, H, W, K = 32, 4, 256, 256, 500
HW = H * W
KP = 512
NJ = KP * C
NCHUNK = NJ // 16

_NC = 2
_NS = 16
NW = _NC * _NS


def _sc_partials(fsum, ind, mask_f, tgt):
    mesh = plsc.VectorSubcoreMesh(core_axis_name="c", subcore_axis_name="s")

    @functools.partial(
        pl.kernel,
        mesh=mesh,
        out_type=[
            jax.ShapeDtypeStruct((NW, 16), jnp.float32),
            jax.ShapeDtypeStruct((NW, 16), jnp.float32),
        ],
        scratch_types=[
            pltpu.VMEM((KP,), jnp.int32),
            pltpu.VMEM((KP,), jnp.float32),
            pltpu.VMEM((NJ,), jnp.float32),
            pltpu.VMEM((NJ,), jnp.int32),
            pltpu.VMEM((NJ,), jnp.float32),
            pltpu.VMEM((16,), jnp.float32),
            pltpu.VMEM((16,), jnp.float32),
            pltpu.SemaphoreType.DMA,
            pltpu.SemaphoreType.DMA,
            pltpu.SemaphoreType.DMA,
            pltpu.SemaphoreType.DMA,
        ],
    )
    def k(f_hbm, ind_hbm, mask_hbm, tgt_hbm, acc_out, ms_out,
          ind_v, mask_v, tgt_v, idx_v, p_v, accv, msv,
          semi, semm, semt, semg):
        wid = lax.axis_index("s") * _NC + lax.axis_index("c")
        b = wid

        cpi = pltpu.async_copy(ind_hbm.at[b], ind_v, semi)
        cpm = pltpu.async_copy(mask_hbm.at[b], mask_v, semm)
        cpt = pltpu.async_copy(tgt_hbm.at[b], tgt_v, semt)
        cpi.wait()

        base = b * (C * HW)

        def build(g, _):
            indv = ind_v[pl.ds(g * 16, 16)] + base
            for c in range(C):
                idx_v[pl.ds(c * KP + g * 16, 16)] = indv + c * HW
            return 0

        lax.fori_loop(0, KP // 16, build, 0)

        cpg = pltpu.async_copy(f_hbm.at[idx_v], p_v, semg)
        cpm.wait()
        cpt.wait()
        cpg.wait()

        def comp(t, carry):
            acc, ms = carry
            m = mask_v[pl.ds(lax.rem(t, KP // 16) * 16, 16)]
            sl = pl.ds(t * 16, 16)
            e = p_v[sl] - tgt_v[sl]
            return acc + (m * e) * e, ms + m

        zero = jnp.zeros((16,), jnp.float32)
        acc, ms = lax.fori_loop(0, NCHUNK, comp, (zero, zero))
        accv[:] = acc
        msv[:] = ms
        pltpu.sync_copy(accv, acc_out.at[b])
        pltpu.sync_copy(msv, ms_out.at[b])

    return k(fsum, ind, mask_f, tgt)


def _tc_reduce(acc, ms):
    def k(acc_ref, ms_ref, out_ref):
        s1 = jnp.sum(acc_ref[...])
        s2 = jnp.sum(ms_ref[...])
        out_ref[0] = s1 / (s2 + 0.0001)

    return pl.pallas_call(
        k,
        out_shape=jax.ShapeDtypeStruct((1,), jnp.float32),
        out_specs=pl.BlockSpec(memory_space=pltpu.SMEM),
    )(acc, ms)


def kernel(output_stage_one, output_stage_two, mask, ind, target):
    fsum = (output_stage_one + output_stage_two).reshape(-1)
    ind_p = jnp.zeros((B, KP), jnp.int32).at[:, :K].set(ind.astype(jnp.int32))
    mask_f = jnp.zeros((B, KP), jnp.float32).at[:, :K].set(
        mask.astype(jnp.float32))
    tgt_p = jnp.zeros((B, C, KP), jnp.float32).at[:, :, :K].set(
        jnp.transpose(target, (0, 2, 1)))
    tgt_flat = tgt_p.reshape(B, NJ)

    acc, ms = _sc_partials(fsum, ind_p, mask_f, tgt_flat)
    return _tc_reduce(acc, ms)[0]
